# initial kernel scaffold (unmeasured)
import jax
import jax.numpy as jnp
from jax import lax
from jax.experimental import pallas as pl
from jax.experimental.pallas import tpu as pltpu

N_DEV = 4


def kernel(x, gamma):
    m, n_per = x.shape
    n_global = n_per * N_DEV
    eps = 1e-5
    g2 = gamma.reshape(1, n_per)

    def body(x_ref, g_ref, out_ref, own_ref, comm_ref, send_sems, recv_sems):
        my = lax.axis_index("i")

        xf = x_ref[:, :]
        partial = jnp.sum(xf * xf, axis=1)
        own_ref[:, :] = partial.reshape(8, 128)

        barrier_sem = pltpu.get_barrier_semaphore()
        for d in range(1, N_DEV):
            pl.semaphore_signal(
                barrier_sem, inc=1,
                device_id=((my + d) % N_DEV,),
                device_id_type=pl.DeviceIdType.MESH,
            )
        pl.semaphore_wait(barrier_sem, N_DEV - 1)

        rdmas = []
        for d in range(1, N_DEV):
            rdma = pltpu.make_async_remote_copy(
                src_ref=own_ref,
                dst_ref=comm_ref.at[(N_DEV - 1) - d],
                send_sem=send_sems.at[d - 1],
                recv_sem=recv_sems.at[(N_DEV - 1) - d],
                device_id=((my + d) % N_DEV,),
                device_id_type=pl.DeviceIdType.MESH,
            )
            rdma.start()
            rdmas.append(rdma)
        for rdma in rdmas:
            rdma.wait()

        tot = own_ref[:, :] + comm_ref[0] + comm_ref[1] + comm_ref[2]
        rstd = lax.rsqrt(tot * (1.0 / n_global) + eps)
        out_ref[:, :] = xf * g_ref[0, :] * rstd.reshape(m, 1)

    return pl.pallas_call(
        body,
        out_shape=jax.ShapeDtypeStruct((m, n_per), jnp.float32),
        in_specs=[
            pl.BlockSpec(memory_space=pltpu.VMEM),
            pl.BlockSpec(memory_space=pltpu.VMEM),
        ],
        out_specs=pl.BlockSpec(memory_space=pltpu.VMEM),
        scratch_shapes=[
            pltpu.VMEM((8, 128), jnp.float32),
            pltpu.VMEM((3, 8, 128), jnp.float32),
            pltpu.SemaphoreType.DMA((3,)),
            pltpu.SemaphoreType.DMA((3,)),
        ],
        compiler_params=pltpu.CompilerParams(collective_id=0),
    )(x, g2)


# baseline (device time: 20220 ns/iter reference)
import jax
import jax.numpy as jnp
from jax import lax
from jax.experimental import pallas as pl
from jax.experimental.pallas import tpu as pltpu

N_DEV = 4


def kernel(x, gamma):
    m, n_per = x.shape
    n_global = n_per * N_DEV
    eps = 1e-5
    g2 = gamma.reshape(1, n_per)

    def body(x_ref, g_ref, out_ref, own_ref, comm_ref, send_sems, recv_sems):
        my = lax.axis_index("i")

        xf = x_ref[:, :]
        own_ref[:, :] = jnp.sum(xf * xf, axis=1, keepdims=True)

        barrier_sem = pltpu.get_barrier_semaphore()
        for d in range(1, N_DEV):
            pl.semaphore_signal(
                barrier_sem, inc=1,
                device_id=((my + d) % N_DEV,),
                device_id_type=pl.DeviceIdType.MESH,
            )
        pl.semaphore_wait(barrier_sem, N_DEV - 1)

        rdmas = []
        for d in range(1, N_DEV):
            rdma = pltpu.make_async_remote_copy(
                src_ref=own_ref,
                dst_ref=comm_ref.at[(N_DEV - 1) - d],
                send_sem=send_sems.at[d - 1],
                recv_sem=recv_sems.at[(N_DEV - 1) - d],
                device_id=((my + d) % N_DEV,),
                device_id_type=pl.DeviceIdType.MESH,
            )
            rdma.start()
            rdmas.append(rdma)
        for rdma in rdmas:
            rdma.wait()

        tot = own_ref[:, :] + comm_ref[0] + comm_ref[1] + comm_ref[2]
        rstd = lax.rsqrt(tot * (1.0 / n_global) + eps)
        out_ref[:, :] = xf * g_ref[0, :] * rstd

    return pl.pallas_call(
        body,
        out_shape=jax.ShapeDtypeStruct((m, n_per), jnp.float32),
        in_specs=[
            pl.BlockSpec(memory_space=pltpu.VMEM),
            pl.BlockSpec(memory_space=pltpu.VMEM),
        ],
        out_specs=pl.BlockSpec(memory_space=pltpu.VMEM),
        scratch_shapes=[
            pltpu.VMEM((m, 1), jnp.float32),
            pltpu.VMEM((3, m, 1), jnp.float32),
            pltpu.SemaphoreType.DMA((3,)),
            pltpu.SemaphoreType.DMA((3,)),
        ],
        compiler_params=pltpu.CompilerParams(collective_id=0),
    )(x, g2)


# device time: 8836 ns/iter; 2.2884x vs baseline; 2.2884x over previous
import jax
import jax.numpy as jnp
from jax import lax
from jax.experimental import pallas as pl
from jax.experimental.pallas import tpu as pltpu

N_DEV = 4


def kernel(x, gamma):
    m, n_per = x.shape
    n_global = n_per * N_DEV
    eps = 1e-5
    g2 = gamma.reshape(1, n_per)

    def body(x_ref, g_ref, out_ref, own_ref, comm_ref, send_sems, recv_sems):
        my = lax.axis_index("i")

        barrier_sem = pltpu.get_barrier_semaphore()
        for d in range(1, N_DEV):
            pl.semaphore_signal(
                barrier_sem, inc=1,
                device_id=((my + d) % N_DEV,),
                device_id_type=pl.DeviceIdType.MESH,
            )

        xf = x_ref[:, :]
        own_ref[:, :] = jnp.sum(xf * xf, axis=1).reshape(8, 128)

        pl.semaphore_wait(barrier_sem, N_DEV - 1)

        rdmas = []
        for d in range(1, N_DEV):
            rdma = pltpu.make_async_remote_copy(
                src_ref=own_ref,
                dst_ref=comm_ref.at[(N_DEV - 1) - d],
                send_sem=send_sems.at[d - 1],
                recv_sem=recv_sems.at[(N_DEV - 1) - d],
                device_id=((my + d) % N_DEV,),
                device_id_type=pl.DeviceIdType.MESH,
            )
            rdma.start()
            rdmas.append(rdma)
        for rdma in rdmas:
            rdma.wait()

        tot = own_ref[:, :] + comm_ref[0] + comm_ref[1] + comm_ref[2]
        rstd = lax.rsqrt(tot * (1.0 / n_global) + eps)
        ri = lax.broadcasted_iota(jnp.int32, (128, 128), 0)
        ci = lax.broadcasted_iota(jnp.int32, (128, 128), 1)
        ident = jnp.where(ri == ci, 1.0, 0.0).astype(jnp.float32)
        g_row = g_ref[0, :]
        for i in range(m // 128):
            col = lax.dot_general(
                ident, rstd[i:i + 1, :],
                (((1,), (1,)), ((), ())),
                preferred_element_type=jnp.float32,
            )
            blk = x_ref[pl.ds(i * 128, 128), :]
            out_ref[pl.ds(i * 128, 128), :] = blk * g_row * col

    return pl.pallas_call(
        body,
        out_shape=jax.ShapeDtypeStruct((m, n_per), jnp.float32),
        in_specs=[
            pl.BlockSpec(memory_space=pltpu.VMEM),
            pl.BlockSpec(memory_space=pltpu.VMEM),
        ],
        out_specs=pl.BlockSpec(memory_space=pltpu.VMEM),
        scratch_shapes=[
            pltpu.VMEM((8, 128), jnp.float32),
            pltpu.VMEM((3, 8, 128), jnp.float32),
            pltpu.SemaphoreType.DMA((3,)),
            pltpu.SemaphoreType.DMA((3,)),
        ],
        compiler_params=pltpu.CompilerParams(collective_id=0),
    )(x, g2)
